# baseline (speedup 1.0000x reference)
import functools

import jax
import jax.numpy as jnp
from jax import lax
from jax.experimental import pallas as pl
from jax.experimental.pallas import tpu as pltpu
from jax.experimental.pallas import tpu_sc as plsc

BATCH = 4096
SEQ = 200
DIM = 64
NUM_CLASSES = 5
VOCAB = 1000000
NC = 2
NS = 16
NW = NC * NS
B_PER_W = BATCH // NW
SPLIT = 104
CPAD = 128
WPR = DIM // 2
WROW = WPR + 1

VBLK = 256
VMAIN = (VOCAB // VBLK) * VBLK
NBLK = VMAIN // VBLK
VTAIL = VOCAB - VMAIN


def _make_relayout_call():
  mesh = plsc.VectorSubcoreMesh(core_axis_name="c", subcore_axis_name="s")

  @functools.partial(
      pl.kernel,
      mesh=mesh,
      compiler_params=pltpu.CompilerParams(
          use_tc_tiling_on_sc=True, needs_layout_passes=False),
      out_type=jax.ShapeDtypeStruct((VOCAB * WROW,), jnp.int32),
      scratch_types=[
          pltpu.VMEM((DIM, VBLK), jnp.float32),
          pltpu.VMEM((DIM, VBLK), jnp.float32),
          pltpu.VMEM((DIM, VBLK), jnp.float32),
          pltpu.VMEM((DIM, VBLK), jnp.float32),
          pltpu.VMEM((VBLK * WROW,), jnp.int32),
          pltpu.VMEM((VBLK * WROW,), jnp.int32),
          pltpu.VMEM((VTAIL * WROW,), jnp.int32),
          pltpu.SemaphoreType.DMA,
          pltpu.SemaphoreType.DMA,
          pltpu.SemaphoreType.DMA,
          pltpu.SemaphoreType.DMA,
          pltpu.SemaphoreType.DMA,
      ],
  )
  def relayout(tt_hbm, tail_hbm, out_hbm,
               in0, in1, in2, in3, ob0, ob1, tailb,
               semi0, semi1, semi2, semi3, semo):
    wid = lax.axis_index("s") * NC + lax.axis_index("c")
    per = NBLK // NW
    extra = NBLK - per * NW
    nblk = jnp.where(wid < extra, per + 1, per)
    b0 = wid * per + jnp.minimum(wid, extra)

    @pl.when(wid == 0)
    def _():
      pltpu.sync_copy(tail_hbm, tailb)
      pltpu.sync_copy(tailb, out_hbm.at[pl.ds(VMAIN * WROW, VTAIL * WROW)])

    def issue_in(blk, buf, sem):
      pltpu.async_copy(tt_hbm.at[:, pl.ds(blk * VBLK, VBLK)], buf, sem)

    def wait_in(blk, buf, sem):
      pltpu.make_async_copy(
          tt_hbm.at[:, pl.ds(blk * VBLK, VBLK)], buf, sem).wait()

    iota16 = lax.iota(jnp.int32, 16)
    sidx = iota16 * WROW

    def round_pack(lo, hi):
      return lax.bitwise_or(
          lax.bitwise_and(hi + jnp.int32(0x8000), jnp.int32(-65536)),
          lax.shift_right_logical(lo + jnp.int32(0x8000), jnp.int32(16)))

    def transform(blk, ibuf, obuf):
      @plsc.parallel_loop(0, (VBLK // 16) * WPR, unroll=8)
      def word(i):
        g = lax.shift_right_logical(i, jnp.int32(5))
        k = lax.bitwise_and(i, jnp.int32(WPR - 1))
        lo = plsc.bitcast(ibuf[2 * k, pl.ds(g * 16, 16)], jnp.int32)
        hi = plsc.bitcast(ibuf[2 * k + 1, pl.ds(g * 16, 16)], jnp.int32)
        plsc.store_scatter(
            obuf, [sidx + (g * (16 * WROW) + k)], round_pack(lo, hi))
      pltpu.async_copy(
          obuf, out_hbm.at[pl.ds(blk * (VBLK * WROW), VBLK * WROW)], semo)

    def wait_out_one():
      pltpu.make_async_copy(
          ob0, out_hbm.at[pl.ds(0, VBLK * WROW)], semo).wait()

    NBUF = 4
    ins = ((in0, semi0), (in1, semi1), (in2, semi2), (in3, semi3))
    obs = (ob0, ob1)

    for p in range(NBUF - 1):
      @pl.when(p < nblk)
      def _():
        issue_in(b0 + p, ins[p][0], ins[p][1])

    def step(i, _):
      for par in range(NBUF):
        j = i + par
        buf, sem = ins[par]
        ibuf, isem = ins[(par + NBUF - 1) % NBUF]

        @pl.when(j < nblk)
        def _():
          @pl.when(j + NBUF - 1 < nblk)
          def _():
            issue_in(b0 + j + NBUF - 1, ibuf, isem)

          wait_in(b0 + j, buf, sem)

          @pl.when(j >= 2)
          def _():
            wait_out_one()

          transform(b0 + j, buf, obs[par % 2])
      return 0

    nquarter = (per + 1 + NBUF - 1) // NBUF
    lax.fori_loop(0, nquarter, lambda i, c: step(i * NBUF, c), 0)

    wait_out_one()
    wait_out_one()

  return relayout


def _make_gather_call():
  mesh = plsc.VectorSubcoreMesh(core_axis_name="c", subcore_axis_name="s")

  @functools.partial(
      pl.kernel,
      mesh=mesh,
      compiler_params=pltpu.CompilerParams(
          use_tc_tiling_on_sc=False, needs_layout_passes=False),
      out_type=jax.ShapeDtypeStruct((BATCH, DIM), jnp.float32),
      scratch_types=[
          pltpu.VMEM((B_PER_W * SEQ,), jnp.int32),
          pltpu.VMEM((SEQ, WROW), jnp.int32),
          pltpu.VMEM((SEQ, WROW), jnp.int32),
          pltpu.VMEM((B_PER_W, DIM), jnp.float32),
          pltpu.SemaphoreType.DMA,
          pltpu.SemaphoreType.DMA,
      ],
  )
  def sc_gather(x_hbm, table_hbm, out_hbm,
                x_v, buf0, buf1, out_v, sem0, sem1):
    wid = lax.axis_index("s") * NC + lax.axis_index("c")
    base = wid * B_PER_W

    pltpu.sync_copy(x_hbm.at[pl.ds(base * SEQ, B_PER_W * SEQ)], x_v)

    def issue(row, buf, sem):
      pltpu.async_copy(
          table_hbm.at[x_v.at[pl.ds(row * SEQ, SPLIT)]],
          buf.at[pl.ds(0, SPLIT)], sem)
      pltpu.async_copy(
          table_hbm.at[x_v.at[pl.ds(row * SEQ + SPLIT, SEQ - SPLIT)]],
          buf.at[pl.ds(SPLIT, SEQ - SPLIT)], sem)

    def wait(row, buf, sem):
      pltpu.make_async_copy(
          table_hbm.at[x_v.at[pl.ds(row * SEQ, SPLIT)]],
          buf.at[pl.ds(0, SPLIT)], sem).wait()
      pltpu.make_async_copy(
          table_hbm.at[x_v.at[pl.ds(row * SEQ + SPLIT, SEQ - SPLIT)]],
          buf.at[pl.ds(SPLIT, SEQ - SPLIT)], sem).wait()

    def load_widened(buf, i):
      out = []
      for j in range(2):
        ii = buf[i, pl.ds(16 * j, 16)]
        even = plsc.bitcast(lax.shift_left(ii, jnp.int32(16)), jnp.float32)
        odd = plsc.bitcast(
            lax.bitwise_and(ii, jnp.int32(-65536)), jnp.float32)
        out += [even, odd]
      return tuple(out)

    def compute(row, buf):
      acc = load_widened(buf, 0)

      def mx(i, a):
        w = load_widened(buf, i)
        return tuple(jnp.maximum(a[k], w[k]) for k in range(4))

      acc = lax.fori_loop(1, SEQ, mx, acc, unroll=4)
      for k in range(4):
        out_v[row, pl.ds(16 * k, 16)] = acc[k]

    issue(0, buf0, sem0)
    bufs = ((buf0, sem0), (buf1, sem1))

    def step(r, _):
      for par, (buf, sem) in enumerate(bufs):
        row = r + par
        nbuf, nsem = bufs[(par + 1) % 2]

        @pl.when(row + 1 < B_PER_W)
        def _():
          issue(row + 1, nbuf, nsem)

        wait(row, buf, sem)
        compute(row, buf)
      return 0

    lax.fori_loop(0, B_PER_W // 2, lambda i, c: step(i * 2, c), 0)

    pltpu.sync_copy(out_v, out_hbm.at[pl.ds(base, B_PER_W)])

  return sc_gather


_relayout_call = _make_relayout_call()
_gather_call = _make_gather_call()


def _linear_body(m_ref, wt_ref, b_ref, o_ref):
  o_ref[...] = (
      jnp.dot(m_ref[...], wt_ref[...], preferred_element_type=jnp.float32)
      + b_ref[...]
  )


_linear_call = pl.pallas_call(
    _linear_body,
    out_shape=jax.ShapeDtypeStruct((BATCH, CPAD), jnp.float32),
)


@jax.jit
def kernel(x, table, W, b):
  tt = table.T
  tail = jnp.pad(
      lax.bitcast_convert_type(
          table[VMAIN:].astype(jnp.bfloat16).reshape(VTAIL, WPR, 2),
          jnp.int32),
      ((0, 0), (0, WROW - WPR))).reshape(-1)
  tb_flat = _relayout_call(tt, tail)
  tb = tb_flat.reshape(VOCAB, WROW)
  m = _gather_call(x.astype(jnp.int32).reshape(-1), tb)
  perm = jnp.concatenate([
      jnp.arange(0, 32, 2), jnp.arange(1, 32, 2),
      jnp.arange(32, 64, 2), jnp.arange(33, 64, 2)])
  wt_pad = jnp.zeros((DIM, CPAD), jnp.float32).at[:, :NUM_CLASSES].set(
      W.T[perm])
  b_pad = jnp.zeros((CPAD,), jnp.float32).at[:NUM_CLASSES].set(b)
  y = _linear_call(m, wt_pad, b_pad)
  return y[:, :NUM_CLASSES]


# speedup vs baseline: 6.6955x; 6.6955x over previous
import functools

import jax
import jax.numpy as jnp
from jax import lax
from jax.experimental import pallas as pl
from jax.experimental.pallas import tpu as pltpu
from jax.experimental.pallas import tpu_sc as plsc

BATCH = 4096
SEQ = 200
DIM = 64
NUM_CLASSES = 5
VOCAB = 1000000
NC = 2
NS = 16
NW = NC * NS
B_PER_W = BATCH // NW
SPLIT = 104
CPAD = 128
WPR = DIM // 2
WROW = WPR + 1

VBLK = 256
VMAIN = (VOCAB // VBLK) * VBLK
NBLK = VMAIN // VBLK
VTAIL = VOCAB - VMAIN


def _make_relayout_call():
  mesh = plsc.VectorSubcoreMesh(core_axis_name="c", subcore_axis_name="s")

  @functools.partial(
      pl.kernel,
      mesh=mesh,
      compiler_params=pltpu.CompilerParams(
          use_tc_tiling_on_sc=True, needs_layout_passes=False),
      out_type=jax.ShapeDtypeStruct((VOCAB * WPR,), jnp.int32),
      scratch_types=[
          pltpu.VMEM((DIM, VBLK), jnp.float32),
          pltpu.VMEM((DIM, VBLK), jnp.float32),
          pltpu.VMEM((DIM, VBLK), jnp.float32),
          pltpu.VMEM((DIM, VBLK), jnp.float32),
          pltpu.VMEM((VBLK * WROW,), jnp.int32),
          pltpu.VMEM((VBLK * WPR,), jnp.int32),
          pltpu.VMEM((VBLK * WPR,), jnp.int32),
          pltpu.VMEM((VTAIL * WPR,), jnp.int32),
          pltpu.SemaphoreType.DMA,
          pltpu.SemaphoreType.DMA,
          pltpu.SemaphoreType.DMA,
          pltpu.SemaphoreType.DMA,
          pltpu.SemaphoreType.DMA,
      ],
  )
  def relayout(tt_hbm, tail_hbm, out_hbm,
               in0, in1, in2, in3, oskew, ob0, ob1, tailb,
               semi0, semi1, semi2, semi3, semo):
    wid = lax.axis_index("s") * NC + lax.axis_index("c")
    per = NBLK // NW
    extra = NBLK - per * NW
    nblk = jnp.where(wid < extra, per + 1, per)
    b0 = wid * per + jnp.minimum(wid, extra)

    @pl.when(wid == 0)
    def _():
      pltpu.sync_copy(tail_hbm, tailb)
      pltpu.sync_copy(tailb, out_hbm.at[pl.ds(VMAIN * WPR, VTAIL * WPR)])

    def issue_in(blk, buf, sem):
      pltpu.async_copy(tt_hbm.at[:, pl.ds(blk * VBLK, VBLK)], buf, sem)

    def wait_in(blk, buf, sem):
      pltpu.make_async_copy(
          tt_hbm.at[:, pl.ds(blk * VBLK, VBLK)], buf, sem).wait()

    iota16 = lax.iota(jnp.int32, 16)
    sidx = iota16 * WROW

    def round_pack(lo, hi):
      return lax.bitwise_or(
          lax.bitwise_and(hi + jnp.int32(0x8000), jnp.int32(-65536)),
          lax.shift_right_logical(lo + jnp.int32(0x8000), jnp.int32(16)))

    def transform(blk, ibuf, obuf):
      @plsc.parallel_loop(0, (VBLK // 16) * WPR, unroll=8)
      def word(i):
        g = lax.shift_right_logical(i, jnp.int32(5))
        k = lax.bitwise_and(i, jnp.int32(WPR - 1))
        lo = plsc.bitcast(ibuf[2 * k, pl.ds(g * 16, 16)], jnp.int32)
        hi = plsc.bitcast(ibuf[2 * k + 1, pl.ds(g * 16, 16)], jnp.int32)
        plsc.store_scatter(
            oskew, [sidx + (g * (16 * WROW) + k)], round_pack(lo, hi))

      @plsc.parallel_loop(0, VBLK, unroll=8)
      def row(v):
        for h in range(2):
          obuf[pl.ds(v * WPR + 16 * h, 16)] = (
              oskew[pl.ds(v * WROW + 16 * h, 16)])

      pltpu.async_copy(
          obuf, out_hbm.at[pl.ds(blk * (VBLK * WPR), VBLK * WPR)], semo)

    def wait_out_one():
      pltpu.make_async_copy(
          ob0, out_hbm.at[pl.ds(0, VBLK * WPR)], semo).wait()

    NBUF = 4
    ins = ((in0, semi0), (in1, semi1), (in2, semi2), (in3, semi3))
    obs = (ob0, ob1)

    for p in range(NBUF - 1):
      @pl.when(p < nblk)
      def _():
        issue_in(b0 + p, ins[p][0], ins[p][1])

    def step(i, _):
      for par in range(NBUF):
        j = i + par
        buf, sem = ins[par]
        ibuf, isem = ins[(par + NBUF - 1) % NBUF]

        @pl.when(j < nblk)
        def _():
          @pl.when(j + NBUF - 1 < nblk)
          def _():
            issue_in(b0 + j + NBUF - 1, ibuf, isem)

          wait_in(b0 + j, buf, sem)

          @pl.when(j >= 2)
          def _():
            wait_out_one()

          transform(b0 + j, buf, obs[par % 2])
      return 0

    nquarter = (per + 1 + NBUF - 1) // NBUF
    lax.fori_loop(0, nquarter, lambda i, c: step(i * NBUF, c), 0)

    wait_out_one()
    wait_out_one()

  return relayout


def _make_gather_call():
  mesh = plsc.VectorSubcoreMesh(core_axis_name="c", subcore_axis_name="s")

  @functools.partial(
      pl.kernel,
      mesh=mesh,
      compiler_params=pltpu.CompilerParams(
          use_tc_tiling_on_sc=False, needs_layout_passes=False),
      out_type=jax.ShapeDtypeStruct((BATCH, DIM), jnp.float32),
      scratch_types=[
          pltpu.VMEM((B_PER_W * SEQ,), jnp.int32),
          pltpu.VMEM((SEQ, WPR), jnp.int32),
          pltpu.VMEM((SEQ, WPR), jnp.int32),
          pltpu.VMEM((B_PER_W, DIM), jnp.float32),
          pltpu.SemaphoreType.DMA,
          pltpu.SemaphoreType.DMA,
      ],
  )
  def sc_gather(x_hbm, table_hbm, out_hbm,
                x_v, buf0, buf1, out_v, sem0, sem1):
    wid = lax.axis_index("s") * NC + lax.axis_index("c")
    base = wid * B_PER_W

    pltpu.sync_copy(x_hbm.at[pl.ds(base * SEQ, B_PER_W * SEQ)], x_v)

    def issue(row, buf, sem):
      pltpu.async_copy(
          table_hbm.at[x_v.at[pl.ds(row * SEQ, SPLIT)]],
          buf.at[pl.ds(0, SPLIT)], sem)
      pltpu.async_copy(
          table_hbm.at[x_v.at[pl.ds(row * SEQ + SPLIT, SEQ - SPLIT)]],
          buf.at[pl.ds(SPLIT, SEQ - SPLIT)], sem)

    def wait(row, buf, sem):
      pltpu.make_async_copy(
          table_hbm.at[x_v.at[pl.ds(row * SEQ, SPLIT)]],
          buf.at[pl.ds(0, SPLIT)], sem).wait()
      pltpu.make_async_copy(
          table_hbm.at[x_v.at[pl.ds(row * SEQ + SPLIT, SEQ - SPLIT)]],
          buf.at[pl.ds(SPLIT, SEQ - SPLIT)], sem).wait()

    def load_widened(buf, i):
      out = []
      for j in range(2):
        ii = buf[i, pl.ds(16 * j, 16)]
        even = plsc.bitcast(lax.shift_left(ii, jnp.int32(16)), jnp.float32)
        odd = plsc.bitcast(
            lax.bitwise_and(ii, jnp.int32(-65536)), jnp.float32)
        out += [even, odd]
      return tuple(out)

    def compute(row, buf):
      acc = load_widened(buf, 0)

      def mx(i, a):
        w = load_widened(buf, i)
        return tuple(jnp.maximum(a[k], w[k]) for k in range(4))

      acc = lax.fori_loop(1, SEQ, mx, acc, unroll=4)
      for k in range(4):
        out_v[row, pl.ds(16 * k, 16)] = acc[k]

    issue(0, buf0, sem0)
    bufs = ((buf0, sem0), (buf1, sem1))

    def step(r, _):
      for par, (buf, sem) in enumerate(bufs):
        row = r + par
        nbuf, nsem = bufs[(par + 1) % 2]

        @pl.when(row + 1 < B_PER_W)
        def _():
          issue(row + 1, nbuf, nsem)

        wait(row, buf, sem)
        compute(row, buf)
      return 0

    lax.fori_loop(0, B_PER_W // 2, lambda i, c: step(i * 2, c), 0)

    pltpu.sync_copy(out_v, out_hbm.at[pl.ds(base, B_PER_W)])

  return sc_gather


_relayout_call = _make_relayout_call()
_gather_call = _make_gather_call()


def _linear_body(m_ref, wt_ref, b_ref, o_ref):
  o_ref[...] = (
      jnp.dot(m_ref[...], wt_ref[...], preferred_element_type=jnp.float32)
      + b_ref[...]
  )


_linear_call = pl.pallas_call(
    _linear_body,
    out_shape=jax.ShapeDtypeStruct((BATCH, CPAD), jnp.float32),
)


@jax.jit
def kernel(x, table, W, b):
  tt = table.T
  tail = lax.bitcast_convert_type(
      table[VMAIN:].astype(jnp.bfloat16).reshape(VTAIL, WPR, 2),
      jnp.int32).reshape(-1)
  tb_flat = _relayout_call(tt, tail)
  tb = tb_flat.reshape(VOCAB, WPR)
  m = _gather_call(x.astype(jnp.int32).reshape(-1), tb)
  perm = jnp.concatenate([
      jnp.arange(0, 32, 2), jnp.arange(1, 32, 2),
      jnp.arange(32, 64, 2), jnp.arange(33, 64, 2)])
  wt_pad = jnp.zeros((DIM, CPAD), jnp.float32).at[:, :NUM_CLASSES].set(
      W.T[perm])
  b_pad = jnp.zeros((CPAD,), jnp.float32).at[:NUM_CLASSES].set(b)
  y = _linear_call(m, wt_pad, b_pad)
  return y[:, :NUM_CLASSES]
